# Initial kernel scaffold; baseline (speedup 1.0000x reference)
import jax
import jax.numpy as jnp
from jax.experimental import pallas as pl


def kernel(memory, input_data, write_weights, query):
    raise NotImplementedError("write your pallas kernel here")


# speedup vs baseline: 1.6640x; 1.6640x over previous
import jax
import jax.numpy as jnp
from jax.experimental import pallas as pl
from jax.experimental.pallas import tpu as pltpu


def _rowsum_scale_body(w_ref, x_ref, xs_ref, acc_ref):
    j = pl.program_id(1)

    @pl.when(j == 0)
    def _():
        acc_ref[...] = jnp.zeros_like(acc_ref)

    acc_ref[...] += jnp.sum(w_ref[...], axis=1, keepdims=True)

    @pl.when(j == pl.num_programs(1) - 1)
    def _():
        xs_ref[...] = x_ref[...] / acc_ref[...]


def _update_body(w_ref, xs_ref, mem_ref, out_ref):
    upd = jax.lax.dot_general(
        w_ref[...], xs_ref[...],
        dimension_numbers=(((0,), (0,)), ((), ())),
        preferred_element_type=jnp.float32)
    out_ref[...] = mem_ref[...] + upd


def _attn_body(q_ref, mem_ref, out_ref, conf_ref, acc_ref, m_ref, l_ref):
    j = pl.program_id(1)

    @pl.when(j == 0)
    def _():
        acc_ref[...] = jnp.zeros_like(acc_ref)
        m_ref[...] = jnp.full_like(m_ref, -jnp.inf)
        l_ref[...] = jnp.zeros_like(l_ref)

    s = jax.lax.dot_general(
        q_ref[...], mem_ref[...],
        dimension_numbers=(((1,), (1,)), ((), ())),
        preferred_element_type=jnp.float32)
    m_prev = m_ref[...]
    m_cur = jnp.maximum(m_prev, jnp.max(s, axis=1, keepdims=True))
    alpha = jnp.exp(m_prev - m_cur)
    p = jnp.exp(s - m_cur)
    l_ref[...] = l_ref[...] * alpha + jnp.sum(p, axis=1, keepdims=True)
    m_ref[...] = m_cur
    acc_ref[...] = acc_ref[...] * alpha + jax.lax.dot_general(
        p, mem_ref[...],
        dimension_numbers=(((1,), (0,)), ((), ())),
        preferred_element_type=jnp.float32)

    @pl.when(j == pl.num_programs(1) - 1)
    def _():
        linv = 1.0 / l_ref[...]
        out_ref[...] = acc_ref[...] * linv
        conf_ref[...] = linv


def kernel(memory, input_data, write_weights, query):
    M, F = memory.shape
    B = input_data.shape[0]
    f32 = jnp.float32

    BB, BMR = min(512, B), min(8192, M)
    xs = pl.pallas_call(
        _rowsum_scale_body,
        grid=(B // BB, M // BMR),
        in_specs=[pl.BlockSpec((BB, BMR), lambda i, j: (i, j)),
                  pl.BlockSpec((BB, F), lambda i, j: (i, 0))],
        out_specs=pl.BlockSpec((BB, F), lambda i, j: (i, 0)),
        out_shape=jax.ShapeDtypeStruct((B, F), f32),
        scratch_shapes=[pltpu.VMEM((BB, 1), f32)],
        compiler_params=pltpu.CompilerParams(
            dimension_semantics=("parallel", "arbitrary"),
            vmem_limit_bytes=56 * 1024 * 1024),
        name="rowsum_scale",
    )(write_weights, input_data)

    BM = min(1024, M)
    memory_new = pl.pallas_call(
        _update_body,
        grid=(M // BM,),
        in_specs=[pl.BlockSpec((B, BM), lambda i: (0, i)),
                  pl.BlockSpec((B, F), lambda i: (0, 0)),
                  pl.BlockSpec((BM, F), lambda i: (i, 0))],
        out_specs=pl.BlockSpec((BM, F), lambda i: (i, 0)),
        out_shape=jax.ShapeDtypeStruct((M, F), f32),
        compiler_params=pltpu.CompilerParams(
            dimension_semantics=("parallel",),
            vmem_limit_bytes=56 * 1024 * 1024),
        name="memory_update",
    )(write_weights, xs, memory)

    BQ, BK = min(512, B), min(2048, M)
    retrieved, conf = pl.pallas_call(
        _attn_body,
        grid=(B // BQ, M // BK),
        in_specs=[pl.BlockSpec((BQ, F), lambda i, j: (i, 0)),
                  pl.BlockSpec((BK, F), lambda i, j: (j, 0))],
        out_specs=[pl.BlockSpec((BQ, F), lambda i, j: (i, 0)),
                   pl.BlockSpec((BQ, 1), lambda i, j: (i, 0))],
        out_shape=[jax.ShapeDtypeStruct((B, F), f32),
                   jax.ShapeDtypeStruct((B, 1), f32)],
        scratch_shapes=[pltpu.VMEM((BQ, F), f32),
                        pltpu.VMEM((BQ, 1), f32),
                        pltpu.VMEM((BQ, 1), f32)],
        compiler_params=pltpu.CompilerParams(
            dimension_semantics=("parallel", "arbitrary"),
            vmem_limit_bytes=56 * 1024 * 1024),
        name="attention_read",
    )(query, memory_new)

    return retrieved, conf[:, 0], memory_new
